# Initial kernel scaffold; baseline (speedup 1.0000x reference)
import jax
import jax.numpy as jnp
from jax.experimental import pallas as pl


def kernel(logits, actions):
    raise NotImplementedError("write your pallas kernel here")


# speedup vs baseline: 1.0272x; 1.0272x over previous
import jax
import jax.numpy as jnp
from jax import lax
from jax.experimental import pallas as pl
from jax.experimental.pallas import tpu as pltpu

B = 128
V = 100000
CW = 2048
NB = (V + CW - 1) // CW

_BIG = 2**30


def _tc_body(act_ref, x_ref, lp_ref, mode_ref, m_ref, s_ref, g_ref, ai_ref):
    j = pl.program_id(0)

    @pl.when(j == 0)
    def _init():
        m_ref[...] = jnp.full((B, 1), -jnp.inf, jnp.float32)
        s_ref[...] = jnp.zeros((B, 1), jnp.float32)
        g_ref[...] = jnp.zeros((B, 1), jnp.float32)
        ai_ref[...] = jnp.zeros((B, 1), jnp.int32)

    x = x_ref[...]
    col0 = j * CW
    cols = col0 + lax.broadcasted_iota(jnp.int32, (B, CW), 1)
    valid = cols < V
    xv = jnp.where(valid, x, -jnp.inf)

    bm = jnp.max(xv, axis=1, keepdims=True)
    bi = jnp.min(jnp.where(xv == bm, cols, _BIG), axis=1, keepdims=True)

    m_old = m_ref[...]
    m_new = jnp.maximum(m_old, bm)
    e = jnp.where(valid, jnp.exp(x - m_new), 0.0)
    s_ref[...] = s_ref[...] * jnp.exp(m_old - m_new) + jnp.sum(
        e, axis=1, keepdims=True)
    ai_ref[...] = jnp.where(bm > m_old, bi, ai_ref[...])
    m_ref[...] = m_new

    a = act_ref[...]
    g_ref[...] += jnp.sum(jnp.where(cols == a, x, 0.0), axis=1, keepdims=True)

    @pl.when(j == NB - 1)
    def _fin():
        lp_ref[...] = (g_ref[...] - m_ref[...] - jnp.log(s_ref[...]))[:, 0]
        mode_ref[...] = ai_ref[...][:, 0]


def kernel(logits, actions):
    a = actions.astype(jnp.int32)
    lp, mode = pl.pallas_call(
        _tc_body,
        grid=(NB,),
        in_specs=[
            pl.BlockSpec((B, 1), lambda j: (0, 0)),
            pl.BlockSpec((B, CW), lambda j: (0, j)),
        ],
        out_specs=[
            pl.BlockSpec((B,), lambda j: (0,)),
            pl.BlockSpec((B,), lambda j: (0,)),
        ],
        out_shape=[
            jax.ShapeDtypeStruct((B,), jnp.float32),
            jax.ShapeDtypeStruct((B,), jnp.int32),
        ],
        scratch_shapes=[
            pltpu.VMEM((B, 1), jnp.float32),
            pltpu.VMEM((B, 1), jnp.float32),
            pltpu.VMEM((B, 1), jnp.float32),
            pltpu.VMEM((B, 1), jnp.int32),
        ],
    )(a, logits)
    return lp, mode
